# baseline (speedup 1.0000x reference)
import functools

import jax
import jax.numpy as jnp
from jax import lax
from jax.experimental import pallas as pl
from jax.experimental.pallas import tpu as pltpu
from jax.experimental.pallas import tpu_sc as plsc

_NC = 2
_NS = 16
_LANES = 16


@functools.lru_cache(maxsize=None)
def _make_scatter(n, d, e_pad):
    nw = _NC * _NS
    chunk = 80
    nbuf = 4
    assert e_pad % (nw * chunk) == 0
    cpw = e_pad // (nw * chunk)
    epw = cpw * chunk
    assert cpw % nbuf == 0 and epw % 8 == 0
    rpt = (n // _NS) // 8 * 8
    rextra = n - _NS * rpt
    assert rextra % 8 == 0
    zrows = 16
    assert rpt % zrows == 0 and rextra <= zrows
    nacc = n + 8
    mesh = plsc.VectorSubcoreMesh(core_axis_name="c", subcore_axis_name="s")

    @functools.partial(
        pl.kernel,
        mesh=mesh,
        out_type=jax.ShapeDtypeStruct((_NC, n, d), jnp.float32),
        scratch_types=(
            [pltpu.VMEM((zrows, d), jnp.float32)]
            + [pltpu.VMEM((chunk,), jnp.int32) for _ in range(nbuf)]
            + [pltpu.VMEM((chunk,), jnp.int32) for _ in range(nbuf)]
            + [pltpu.VMEM((chunk, d), jnp.float32) for _ in range(nbuf)]
            + [pltpu.SemaphoreType.DMA for _ in range(3 * nbuf)]
            + [pltpu.VMEM_SHARED((nacc, d), jnp.float32)]
        ),
    )
    def scatter_kernel(h_hbm, src_hbm, dst_hbm, out_hbm, zbuf, *rest):
        srcb = rest[0:nbuf]
        dstb = rest[nbuf:2 * nbuf]
        rows = rest[2 * nbuf:3 * nbuf]
        isem = rest[3 * nbuf:4 * nbuf]
        gsem = rest[4 * nbuf:5 * nbuf]
        ssem = rest[5 * nbuf:6 * nbuf]
        acc_sh = rest[6 * nbuf]
        cid = lax.axis_index("c")
        sid = lax.axis_index("s")
        wid = sid * _NC + cid
        e0 = wid * epw

        def idx_issue(b, t):
            pltpu.async_copy(src_hbm.at[pl.ds(e0 + t * chunk, chunk)], srcb[b], isem[b])
            pltpu.async_copy(dst_hbm.at[pl.ds(e0 + t * chunk, chunk)], dstb[b], isem[b])

        def idx_wait(b, t):
            pltpu.make_async_copy(src_hbm.at[pl.ds(e0 + t * chunk, chunk)],
                                  srcb[b], isem[b]).wait()
            pltpu.make_async_copy(dst_hbm.at[pl.ds(e0 + t * chunk, chunk)],
                                  dstb[b], isem[b]).wait()

        def gather_issue(b):
            pltpu.async_copy(h_hbm.at[srcb[b]], rows[b], gsem[b])

        def gather_wait(b):
            pltpu.make_async_copy(h_hbm.at[srcb[b]], rows[b], gsem[b]).wait()

        def scatter_issue(b):
            pltpu.async_copy(rows[b], acc_sh.at[dstb[b]], ssem[b], add=True)

        def scatter_wait(b):
            pltpu.make_async_copy(rows[b], acc_sh.at[dstb[b]], ssem[b]).wait()

        idx_issue(0, 0)
        idx_issue(1, 1)
        idx_wait(0, 0)
        gather_issue(0)

        def zstore(t, carry):
            r = t // (d // _LANES)
            c16 = (t % (d // _LANES)) * _LANES
            zbuf[r, pl.ds(c16, _LANES)] = jnp.zeros((_LANES,), jnp.float32)
            return carry
        lax.fori_loop(0, zrows * (d // _LANES), zstore, 0)
        row0 = sid * rpt
        def zcopy(j, carry):
            pltpu.sync_copy(zbuf, acc_sh.at[pl.ds(row0 + j * zrows, zrows)])
            return carry
        lax.fori_loop(0, rpt // zrows, zcopy, 0)
        @pl.when(sid == _NS - 1)
        def _ztail():
            pltpu.sync_copy(zbuf.at[pl.ds(0, rextra)],
                            acc_sh.at[pl.ds(_NS * rpt, rextra)])
        plsc.subcore_barrier()

        def step(i, carry):
            for b in range(nbuf):
                t = i * nbuf + b
                b2 = (b + 2) % nbuf
                b1 = (b + 1) % nbuf
                @pl.when(t + 2 < cpw)
                def _prefetch():
                    @pl.when(t >= 2)
                    def _drain():
                        scatter_wait(b2)
                    idx_issue(b2, t + 2)
                @pl.when(t + 1 < cpw)
                def _gnext():
                    idx_wait(b1, t + 1)
                    gather_issue(b1)
                gather_wait(b)
                scatter_issue(b)
            return carry
        lax.fori_loop(0, cpw // nbuf, step, 0)
        for b in range(nbuf):
            scatter_wait(b)
        plsc.subcore_barrier()

        pltpu.sync_copy(acc_sh.at[pl.ds(row0, rpt)], out_hbm.at[cid].at[pl.ds(row0, rpt)])
        @pl.when(sid == _NS - 1)
        def _():
            pltpu.sync_copy(acc_sh.at[pl.ds(_NS * rpt, rextra)],
                            out_hbm.at[cid].at[pl.ds(_NS * rpt, rextra)])

    return scatter_kernel


@functools.lru_cache(maxsize=None)
def _make_dense(n, d_in, d, relu_out):
    def body(h_ref, a0_ref, a1_ref, w1_ref, b1_ref, w2_ref, b2_ref,
             g_ref, bt_ref, o_ref):
        z = h_ref[...] + a0_ref[...] + a1_ref[...]
        z = jnp.dot(z, w1_ref[...], preferred_element_type=jnp.float32) + b1_ref[...]
        z = jnp.maximum(z, 0.0)
        z = jnp.dot(z, w2_ref[...], preferred_element_type=jnp.float32) + b2_ref[...]
        mu = jnp.mean(z, axis=0, keepdims=True)
        var = jnp.mean((z - mu) * (z - mu), axis=0, keepdims=True)
        z = g_ref[...] * (z - mu) * lax.rsqrt(var + 1e-5) + bt_ref[...]
        if relu_out:
            z = jnp.maximum(z, 0.0)
        o_ref[...] = z

    return pl.pallas_call(
        body,
        out_shape=jax.ShapeDtypeStruct((n, d), jnp.float32),
    )


def kernel(nodes, edge_indexs, graph_indicators,
           W1_0, b1_0, W2_0, b2_0, gamma_0, beta_0,
           W1_1, b1_1, W2_1, b2_1, gamma_1, beta_1,
           W1_2, b1_2, W2_2, b2_2, gamma_2, beta_2):
    del graph_indicators
    n, d = nodes.shape
    e = edge_indexs.shape[1]
    chunk, nw, nbuf = 80, _NC * _NS, 4
    gran = chunk * nw * nbuf
    e_pad = -(-e // gran) * gran
    src = jnp.concatenate([edge_indexs[0], jnp.zeros((e_pad - e,), jnp.int32)])
    dst = jnp.concatenate([edge_indexs[1], jnp.full((e_pad - e,), n, jnp.int32)])
    params = [
        (W1_0, b1_0, W2_0, b2_0, gamma_0, beta_0),
        (W1_1, b1_1, W2_1, b2_1, gamma_1, beta_1),
        (W1_2, b1_2, W2_2, b2_2, gamma_2, beta_2),
    ]
    scatter = _make_scatter(n, d, e_pad)
    h = nodes
    for layer, (w1, b1, w2, b2, g, bt) in enumerate(params):
        agg = scatter(h, src, dst)
        dense = _make_dense(n, w1.shape[0], d, layer < len(params) - 1)
        h = dense(h, agg[0], agg[1], w1, b1.reshape(1, d), w2, b2.reshape(1, d),
                  g.reshape(1, d), bt.reshape(1, d))
    return h
